# Initial kernel scaffold; baseline (speedup 1.0000x reference)
import jax
import jax.numpy as jnp
from jax.experimental import pallas as pl


def kernel(x, W_enc, b_enc, embedding, W_dec, b_dec):
    raise NotImplementedError("write your pallas kernel here")


# speedup vs baseline: 1.0287x; 1.0287x over previous
import jax
import jax.numpy as jnp
from jax.experimental import pallas as pl

_N, _C, _HW = 8, 3, 224
_P = 56
_D = 64
_K = 1024
_ROWS = _N * _P * _P
_TILE = 784
_NT = _ROWS // _TILE


def _encvq_body(xp_ref, w_ref, b_ref, e_ref, z_ref, zq_ref):
    xp = xp_ref[...]
    w = w_ref[...]
    b = b_ref[...]
    e = e_ref[...]
    z = jnp.maximum(jnp.dot(xp, w, preferred_element_type=jnp.float32) + b, 0.0)
    sc = jax.lax.dot_general(z, e, (((1,), (1,)), ((), ())),
                             preferred_element_type=jnp.float32)
    ones = jnp.ones((1, _D), jnp.float32)
    en = jax.lax.dot_general(ones, e * e, (((1,), (1,)), ((), ())),
                             preferred_element_type=jnp.float32)
    rn = jnp.sum(z * z, axis=1, keepdims=True)
    dist = (rn + en) - 2.0 * sc
    m = jnp.min(dist, axis=1, keepdims=True)
    iota = jax.lax.broadcasted_iota(jnp.int32, dist.shape, 1)
    idx = jnp.min(jnp.where(dist == m, iota, jnp.int32(2 ** 30)),
                  axis=1, keepdims=True)
    oh = (iota == idx).astype(jnp.float32)
    zq = jnp.dot(oh, e, preferred_element_type=jnp.float32)
    z_ref[...] = z
    zq_ref[...] = zq


def _dec_body(zt_ref, w_ref, b_ref, o_ref):
    o_ref[0] = jnp.dot(w_ref[...], zt_ref[0],
                       preferred_element_type=jnp.float32) + b_ref[...]


def kernel(x, W_enc, b_enc, embedding, W_dec, b_dec):
    xp = x.reshape(_N, _C, _P, 4, _P, 4).transpose(0, 2, 4, 1, 3, 5)
    xp = xp.reshape(_ROWS, _C * 16)
    w1 = W_enc.reshape(_D, _C * 16).T
    b1 = b_enc.reshape(1, _D)
    w2 = W_dec[:, :, ::-1, ::-1].transpose(1, 0, 2, 3).reshape(_D, _C * 16).T
    b2 = jnp.repeat(b_dec, 16).reshape(_C * 16, 1)

    z_flat, zq_flat = pl.pallas_call(
        _encvq_body,
        grid=(_NT,),
        in_specs=[
            pl.BlockSpec((_TILE, _C * 16), lambda i: (i, 0)),
            pl.BlockSpec((_C * 16, _D), lambda i: (0, 0)),
            pl.BlockSpec((1, _D), lambda i: (0, 0)),
            pl.BlockSpec((_K, _D), lambda i: (0, 0)),
        ],
        out_specs=[
            pl.BlockSpec((_TILE, _D), lambda i: (i, 0)),
            pl.BlockSpec((_TILE, _D), lambda i: (i, 0)),
        ],
        out_shape=[jax.ShapeDtypeStruct((_ROWS, _D), jnp.float32)] * 2,
    )(xp, w1, b1, embedding)

    z_q = zq_flat.reshape(_N, _D, _P, _P)
    z = z_flat.reshape(_N, _P, _P, _D).transpose(0, 3, 1, 2)

    zt = zq_flat.reshape(_N, _D, _P * _P)
    out_t = pl.pallas_call(
        _dec_body,
        grid=(_N,),
        in_specs=[
            pl.BlockSpec((1, _D, _P * _P), lambda n: (n, 0, 0)),
            pl.BlockSpec((_C * 16, _D), lambda n: (0, 0)),
            pl.BlockSpec((_C * 16, 1), lambda n: (0, 0)),
        ],
        out_specs=pl.BlockSpec((1, _C * 16, _P * _P), lambda n: (n, 0, 0)),
        out_shape=jax.ShapeDtypeStruct((_N, _C * 16, _P * _P), jnp.float32),
    )(zt, w2, b2)

    out = out_t.reshape(_N, _C, 4, 4, _P, _P).transpose(0, 1, 4, 2, 5, 3)
    out = out.reshape(_N, _C, _HW, _HW)
    return out, z, z_q


# speedup vs baseline: 1.1758x; 1.1431x over previous
import jax
import jax.numpy as jnp
from jax.experimental import pallas as pl

_N, _C, _HW = 8, 3, 224
_P = 56
_D = 64
_K = 1024
_ROWS = _N * _P * _P
_IMG = _P * _P
_CHUNK = 784
_NCH = _IMG // _CHUNK


def _encvq_body(xp_ref, w_ref, b_ref, e_ref, zt_ref, zq_ref):
    w = w_ref[...]
    b = b_ref[...]
    e = e_ref[...]
    ones = jnp.ones((1, _D), jnp.float32)
    en = jax.lax.dot_general(ones, e * e, (((1,), (1,)), ((), ())),
                             preferred_element_type=jnp.float32)
    z_parts = []
    for c in range(_NCH):
        xp = xp_ref[c * _CHUNK:(c + 1) * _CHUNK, :]
        z = jnp.maximum(jnp.dot(xp, w, preferred_element_type=jnp.float32) + b,
                        0.0)
        sc = jax.lax.dot_general(z, e, (((1,), (1,)), ((), ())),
                                 preferred_element_type=jnp.float32)
        rn = jnp.sum(z * z, axis=1, keepdims=True)
        dist = (rn + en) - 2.0 * sc
        m = jnp.min(dist, axis=1, keepdims=True)
        iota = jax.lax.broadcasted_iota(jnp.int32, dist.shape, 1)
        idx = jnp.min(jnp.where(dist == m, iota, jnp.int32(2 ** 30)),
                      axis=1, keepdims=True)
        oh = (iota == idx).astype(jnp.float32)
        zq = jnp.dot(oh, e, preferred_element_type=jnp.float32)
        zq_ref[c * _CHUNK:(c + 1) * _CHUNK, :] = zq
        z_parts.append(z)
    zfull = jnp.concatenate(z_parts, axis=0)
    zt_ref[0] = zfull.T


def _dec_body(zt_ref, w_ref, b_ref, o_ref):
    ot = jnp.dot(w_ref[...], zt_ref[0],
                 preferred_element_type=jnp.float32) + b_ref[...]
    oimg = ot.reshape(_C, 4, 4, _P, _P).transpose(0, 3, 1, 4, 2)
    o_ref[0] = oimg.reshape(_C, _HW, _HW)


def kernel(x, W_enc, b_enc, embedding, W_dec, b_dec):
    xp = x.reshape(_N, _C, _P, 4, _P, 4).transpose(0, 2, 4, 1, 3, 5)
    xp = xp.reshape(_ROWS, _C * 16)
    w1 = W_enc.reshape(_D, _C * 16).T
    b1 = b_enc.reshape(1, _D)
    w2 = W_dec[:, :, ::-1, ::-1].transpose(1, 0, 2, 3).reshape(_D, _C * 16).T
    b2 = jnp.repeat(b_dec, 16).reshape(_C * 16, 1)

    z_nchw, zq_flat = pl.pallas_call(
        _encvq_body,
        grid=(_N,),
        in_specs=[
            pl.BlockSpec((_IMG, _C * 16), lambda i: (i, 0)),
            pl.BlockSpec((_C * 16, _D), lambda i: (0, 0)),
            pl.BlockSpec((1, _D), lambda i: (0, 0)),
            pl.BlockSpec((_K, _D), lambda i: (0, 0)),
        ],
        out_specs=[
            pl.BlockSpec((1, _D, _IMG), lambda i: (i, 0, 0)),
            pl.BlockSpec((_IMG, _D), lambda i: (i, 0)),
        ],
        out_shape=[
            jax.ShapeDtypeStruct((_N, _D, _IMG), jnp.float32),
            jax.ShapeDtypeStruct((_ROWS, _D), jnp.float32),
        ],
    )(xp, w1, b1, embedding)

    z_q = zq_flat.reshape(_N, _D, _P, _P)
    z = z_nchw.reshape(_N, _D, _P, _P)

    zt = zq_flat.reshape(_N, _D, _IMG)
    out_t = pl.pallas_call(
        _dec_body,
        grid=(_N,),
        in_specs=[
            pl.BlockSpec((1, _D, _IMG), lambda n: (n, 0, 0)),
            pl.BlockSpec((_C * 16, _D), lambda n: (0, 0)),
            pl.BlockSpec((_C * 16, 1), lambda n: (0, 0)),
        ],
        out_specs=pl.BlockSpec((1, _C, _HW, _HW), lambda n: (n, 0, 0, 0)),
        out_shape=jax.ShapeDtypeStruct((_N, _C, _HW, _HW), jnp.float32),
    )(zt, w2, b2)

    return out_t, z, z_q


# speedup vs baseline: 1.2220x; 1.0392x over previous
import jax
import jax.numpy as jnp
from jax.experimental import pallas as pl

_N, _C, _HW = 8, 3, 224
_P = 56
_D = 64
_K = 1024
_ROWS = _N * _P * _P
_IMG = _P * _P
_CHUNK = 784
_NCH = _IMG // _CHUNK


def _patchify_body(x_ref, xp_ref):
    xim = x_ref[0]
    xp = xim.reshape(_C, _P, 4, _P, 4).transpose(1, 3, 0, 2, 4)
    xp_ref[0] = xp.reshape(_IMG, _C * 16)


def _encvq_body(xp_ref, w_ref, b_ref, e_ref, zt_ref, zq_ref):
    w = w_ref[...]
    b = b_ref[...]
    e = e_ref[...]
    ones = jnp.ones((1, _D), jnp.float32)
    en = jax.lax.dot_general(ones, e * e, (((1,), (1,)), ((), ())),
                             preferred_element_type=jnp.float32)
    z_parts = []
    for c in range(_NCH):
        xp = xp_ref[c * _CHUNK:(c + 1) * _CHUNK, :]
        z = jnp.maximum(jnp.dot(xp, w, preferred_element_type=jnp.float32) + b,
                        0.0)
        sc = jax.lax.dot_general(z, e, (((1,), (1,)), ((), ())),
                                 preferred_element_type=jnp.float32)
        rn = jnp.sum(z * z, axis=1, keepdims=True)
        dist = (rn + en) - 2.0 * sc
        m = jnp.min(dist, axis=1, keepdims=True)
        iota = jax.lax.broadcasted_iota(jnp.int32, dist.shape, 1)
        idx = jnp.min(jnp.where(dist == m, iota, jnp.int32(2 ** 30)),
                      axis=1, keepdims=True)
        oh = (iota == idx).astype(jnp.float32)
        zq = jnp.dot(oh, e, preferred_element_type=jnp.float32)
        zq_ref[c * _CHUNK:(c + 1) * _CHUNK, :] = zq
        z_parts.append(z)
    zfull = jnp.concatenate(z_parts, axis=0)
    zt_ref[0] = zfull.T


def _dec_body(zt_ref, w_ref, b_ref, o_ref):
    ot = jnp.dot(w_ref[...], zt_ref[0],
                 preferred_element_type=jnp.float32) + b_ref[...]
    oimg = ot.reshape(_C, 4, 4, _P, _P).transpose(0, 3, 1, 4, 2)
    o_ref[0] = oimg.reshape(_C, _HW, _HW)


def kernel(x, W_enc, b_enc, embedding, W_dec, b_dec):
    xp = pl.pallas_call(
        _patchify_body,
        grid=(_N,),
        in_specs=[pl.BlockSpec((1, _C, _HW, _HW), lambda i: (i, 0, 0, 0))],
        out_specs=pl.BlockSpec((1, _IMG, _C * 16), lambda i: (i, 0, 0)),
        out_shape=jax.ShapeDtypeStruct((_N, _IMG, _C * 16), jnp.float32),
    )(x).reshape(_ROWS, _C * 16)
    w1 = W_enc.reshape(_D, _C * 16).T
    b1 = b_enc.reshape(1, _D)
    w2 = W_dec[:, :, ::-1, ::-1].transpose(1, 0, 2, 3).reshape(_D, _C * 16).T
    b2 = jnp.repeat(b_dec, 16).reshape(_C * 16, 1)

    z_nchw, zq_flat = pl.pallas_call(
        _encvq_body,
        grid=(_N,),
        in_specs=[
            pl.BlockSpec((_IMG, _C * 16), lambda i: (i, 0)),
            pl.BlockSpec((_C * 16, _D), lambda i: (0, 0)),
            pl.BlockSpec((1, _D), lambda i: (0, 0)),
            pl.BlockSpec((_K, _D), lambda i: (0, 0)),
        ],
        out_specs=[
            pl.BlockSpec((1, _D, _IMG), lambda i: (i, 0, 0)),
            pl.BlockSpec((_IMG, _D), lambda i: (i, 0)),
        ],
        out_shape=[
            jax.ShapeDtypeStruct((_N, _D, _IMG), jnp.float32),
            jax.ShapeDtypeStruct((_ROWS, _D), jnp.float32),
        ],
    )(xp, w1, b1, embedding)

    z_q = zq_flat.reshape(_N, _D, _P, _P)
    z = z_nchw.reshape(_N, _D, _P, _P)

    zt = zq_flat.reshape(_N, _D, _IMG)
    out_t = pl.pallas_call(
        _dec_body,
        grid=(_N,),
        in_specs=[
            pl.BlockSpec((1, _D, _IMG), lambda n: (n, 0, 0)),
            pl.BlockSpec((_C * 16, _D), lambda n: (0, 0)),
            pl.BlockSpec((_C * 16, 1), lambda n: (0, 0)),
        ],
        out_specs=pl.BlockSpec((1, _C, _HW, _HW), lambda n: (n, 0, 0, 0)),
        out_shape=jax.ShapeDtypeStruct((_N, _C, _HW, _HW), jnp.float32),
    )(zt, w2, b2)

    return out_t, z, z_q


# speedup vs baseline: 1.2696x; 1.0390x over previous
import jax
import jax.numpy as jnp
from jax.experimental import pallas as pl
from jax.experimental.pallas import tpu as pltpu

_N, _C, _HW = 8, 3, 224
_P = 56
_D = 64
_K = 1024
_ROWS = _N * _P * _P
_IMG = _P * _P
_CHUNK = 784
_NCH = _IMG // _CHUNK


def _fused_body(x_ref, w1_ref, b1_ref, e_ref,
                zt_ref, zq_ref, xp_ref):
    w1 = w1_ref[...]
    b1 = b1_ref[...]
    e = e_ref[...]
    ones = jnp.ones((1, _D), jnp.float32)
    en = jax.lax.dot_general(ones, e * e, (((1,), (1,)), ((), ())),
                             preferred_element_type=jnp.float32)
    xim = x_ref[0]
    xp_full = xim.reshape(_C, _P, 4, _P, 4).transpose(1, 3, 0, 2, 4)
    xp_ref[...] = xp_full.reshape(_IMG, _C * 16)
    z_parts, zq_parts = [], []
    for c in range(_NCH):
        xp = xp_ref[c * _CHUNK:(c + 1) * _CHUNK, :]
        z = jnp.maximum(
            jnp.dot(xp, w1, preferred_element_type=jnp.float32) + b1, 0.0)
        sc = jax.lax.dot_general(z, e, (((1,), (1,)), ((), ())),
                                 preferred_element_type=jnp.float32)
        rn = jnp.sum(z * z, axis=1, keepdims=True)
        dist = (rn + en) - 2.0 * sc
        m = jnp.min(dist, axis=1, keepdims=True)
        iota = jax.lax.broadcasted_iota(jnp.int32, dist.shape, 1)
        idx = jnp.min(jnp.where(dist == m, iota, jnp.int32(2 ** 30)),
                      axis=1, keepdims=True)
        oh = (iota == idx).astype(jnp.float32)
        zq = jnp.dot(oh, e, preferred_element_type=jnp.float32)
        z_parts.append(z)
        zq_parts.append(zq)
    zfull = jnp.concatenate(z_parts, axis=0)
    zqfull = jnp.concatenate(zq_parts, axis=0)
    zq_ref[...] = zqfull
    zt_ref[0] = zfull.T


def _dec_body(zt_ref, w_ref, b_ref, o_ref):
    ot = jnp.dot(w_ref[...], zt_ref[0],
                 preferred_element_type=jnp.float32) + b_ref[...]
    oimg = ot.reshape(_C, 4, 4, _P, _P).transpose(0, 3, 1, 4, 2)
    o_ref[0] = oimg.reshape(_C, _HW, _HW)


def kernel(x, W_enc, b_enc, embedding, W_dec, b_dec):
    w1 = W_enc.reshape(_D, _C * 16).T
    b1 = b_enc.reshape(1, _D)
    w2 = W_dec[:, :, ::-1, ::-1].transpose(1, 0, 2, 3).reshape(_D, _C * 16).T
    b2 = jnp.repeat(b_dec, 16).reshape(_C * 16, 1)

    z_nchw, zq_flat = pl.pallas_call(
        _fused_body,
        grid=(_N,),
        in_specs=[
            pl.BlockSpec((1, _C, _HW, _HW), lambda i: (i, 0, 0, 0)),
            pl.BlockSpec((_C * 16, _D), lambda i: (0, 0)),
            pl.BlockSpec((1, _D), lambda i: (0, 0)),
            pl.BlockSpec((_K, _D), lambda i: (0, 0)),
        ],
        out_specs=[
            pl.BlockSpec((1, _D, _IMG), lambda i: (i, 0, 0)),
            pl.BlockSpec((_IMG, _D), lambda i: (i, 0)),
        ],
        out_shape=[
            jax.ShapeDtypeStruct((_N, _D, _IMG), jnp.float32),
            jax.ShapeDtypeStruct((_ROWS, _D), jnp.float32),
        ],
        scratch_shapes=[pltpu.VMEM((_IMG, _C * 16), jnp.float32)],
    )(x, w1, b1, embedding)

    z_q = zq_flat.reshape(_N, _D, _P, _P)
    z = z_nchw.reshape(_N, _D, _P, _P)

    zt = zq_flat.reshape(_N, _D, _IMG)
    out = pl.pallas_call(
        _dec_body,
        grid=(_N,),
        in_specs=[
            pl.BlockSpec((1, _D, _IMG), lambda n: (n, 0, 0)),
            pl.BlockSpec((_C * 16, _D), lambda n: (0, 0)),
            pl.BlockSpec((_C * 16, 1), lambda n: (0, 0)),
        ],
        out_specs=pl.BlockSpec((1, _C, _HW, _HW), lambda n: (n, 0, 0, 0)),
        out_shape=jax.ShapeDtypeStruct((_N, _C, _HW, _HW), jnp.float32),
    )(zt, w2, b2)
    return out, z, z_q
